# Initial kernel scaffold; baseline (speedup 1.0000x reference)
import jax
import jax.numpy as jnp
from jax.experimental import pallas as pl


def kernel(x, edge_index, W1, W2, W3, W4, W5, W6, W7, W8, b1, b2, b3, b4, b5, b6, b7, b8):
    raise NotImplementedError("write your pallas kernel here")


# speedup vs baseline: 6.0803x; 6.0803x over previous
import functools

import jax
import jax.numpy as jnp
from jax import lax
from jax.experimental import pallas as pl
from jax.experimental.pallas import tpu as pltpu
from jax.experimental.pallas import tpu_sc as plsc

N = 10000
E = 320000
NCORES = 2
NSUB = 16
NW = NCORES * NSUB
BLK = 128
NBLK = 80
EPAD = NW * NBLK * BLK
NPAD = 10240
RPT = NPAD // NSUB
LCH = 640

_MESH = plsc.VectorSubcoreMesh(core_axis_name="c", subcore_axis_name="s")


def _deg_kernel(dst2d, zeros1, deg_out, ones_v, dst_v, acc, sem):
    del sem
    cid = lax.axis_index("c")
    sid = lax.axis_index("s")
    w = cid * NSUB + sid

    for i in range(BLK // 16):
        ones_v[pl.ds(i * 16, 16)] = jnp.ones((16,), jnp.float32)
    pltpu.sync_copy(dst2d.at[pl.ds(w * NBLK, NBLK)], dst_v)
    pltpu.sync_copy(zeros1.at[pl.ds(sid * RPT, RPT)],
                    acc.at[pl.ds(sid * RPT, RPT)])
    plsc.subcore_barrier()

    def body(j, carry):
        pltpu.sync_copy(ones_v, acc.at[dst_v.at[j]], add=True)
        return carry

    lax.fori_loop(0, NBLK, body, 0)
    plsc.subcore_barrier()
    pltpu.sync_copy(acc.at[pl.ds(sid * RPT, RPT)],
                    deg_out.at[pl.ds(cid * NPAD + sid * RPT, RPT)])


_deg = functools.partial(
    pl.kernel,
    out_type=jax.ShapeDtypeStruct((NCORES * NPAD,), jnp.float32),
    mesh=_MESH,
    scratch_types=[
        pltpu.VMEM((BLK,), jnp.float32),
        pltpu.VMEM((NBLK, BLK), jnp.int32),
        pltpu.VMEM_SHARED((NPAD,), jnp.float32),
        pltpu.SemaphoreType.DMA,
    ],
)(_deg_kernel)


def _make_scatter(pd):
    @functools.partial(
        pl.kernel,
        out_type=jax.ShapeDtypeStruct((NCORES * pd * NPAD,), jnp.float32),
        mesh=_MESH,
        scratch_types=[
            pltpu.VMEM((NBLK, BLK), jnp.int32),
            pltpu.VMEM((NBLK, BLK), jnp.int32),
            [pltpu.VMEM((BLK,), jnp.float32) for _ in range(pd)],
            [pltpu.VMEM_SHARED((NPAD,), jnp.float32) for _ in range(pd)],
            [pltpu.VMEM_SHARED((NPAD,), jnp.float32) for _ in range(pd)],
            pltpu.SemaphoreType.DMA,
        ],
    )
    def scatter_kernel(src2d, dst2d, u_fm, zeros1, out_hbm,
                       src_v, dst_v, colb, u_sc, acc, sem0):
        cid = lax.axis_index("c")
        sid = lax.axis_index("s")
        w = cid * NSUB + sid

        pltpu.sync_copy(src2d.at[pl.ds(w * NBLK, NBLK)], src_v)
        pltpu.sync_copy(dst2d.at[pl.ds(w * NBLK, NBLK)], dst_v)
        for k in range(pd):
            pltpu.sync_copy(u_fm.at[pl.ds(k * NPAD + sid * RPT, RPT)],
                            u_sc[k].at[pl.ds(sid * RPT, RPT)])
            pltpu.sync_copy(zeros1.at[pl.ds(sid * RPT, RPT)],
                            acc[k].at[pl.ds(sid * RPT, RPT)])
        plsc.subcore_barrier()

        def body(j, carry):
            cps = [pltpu.async_copy(u_sc[k].at[src_v.at[j]], colb[k], sem0)
                   for k in range(pd)]
            for cp in cps:
                cp.wait()
            for k in range(pd):
                pltpu.sync_copy(colb[k], acc[k].at[dst_v.at[j]], add=True)
            return carry

        lax.fori_loop(0, NBLK, body, 0)
        plsc.subcore_barrier()
        for k in range(pd):
            pltpu.sync_copy(
                acc[k].at[pl.ds(sid * RPT, RPT)],
                out_hbm.at[pl.ds((cid * pd + k) * NPAD + sid * RPT, RPT)])

    return scatter_kernel


_scatter16 = _make_scatter(16)
_scatter32 = _make_scatter(32)


def _dotT(w, h):
    return lax.dot_general(w, h, (((0,), (0,)), ((), ())),
                           preferred_element_type=jnp.float32,
                           precision=lax.Precision.HIGHEST)


def _tc_call(body, shapes, *args):
    out_shape = [jax.ShapeDtypeStruct(s, jnp.float32) for s in shapes]
    return pl.pallas_call(body, out_shape=out_shape)(*args)


def _k0_body(deg_ref, xt_ref, w1_ref, dinv_ref, u_ref):
    deg = deg_ref[0, :, :N] + deg_ref[1, :, :N] + 1.0
    dinv = lax.rsqrt(deg)
    dinv_ref[...] = dinv
    u_ref[...] = dinv * _dotT(w1_ref[...], xt_ref[...])


def _combine(s_ref, u_ref, dinv_ref):
    return dinv_ref[...] * (s_ref[0, :, :N] + s_ref[1, :, :N] + u_ref[...])


def _kB_body(s_ref, u_ref, dinv_ref, b_ref, out_ref):
    h = jax.nn.relu(_combine(s_ref, u_ref, dinv_ref) + b_ref[...])
    out_ref[...] = dinv_ref[...] * h


def _kA_body(s_ref, u_ref, dinv_ref, b_ref, w_ref, out_ref):
    h = jax.nn.relu(_combine(s_ref, u_ref, dinv_ref) + b_ref[...])
    out_ref[...] = dinv_ref[...] * _dotT(w_ref[...], h)


def _kC_body(s_ref, u_ref, dinv_ref, w2_ref, b2_ref, w3_ref, out_ref):
    t = _combine(s_ref, u_ref, dinv_ref)
    h = jax.nn.relu(_dotT(w2_ref[...], t) + b2_ref[...])
    out_ref[...] = dinv_ref[...] * _dotT(w3_ref[...], h)


def _kD_body(s_ref, u_ref, dinv_ref, w8_ref, b8_ref, x2_ref, out_ref):
    t = _combine(s_ref, u_ref, dinv_ref)
    out_ref[...] = _dotT(w8_ref[...], t) + b8_ref[...] + x2_ref[...]


def kernel(x, edge_index, W1, W2, W3, W4, W5, W6, W7, W8,
           b1, b2, b3, b4, b5, b6, b7, b8):
    src, dst = edge_index[0], edge_index[1]
    pad = EPAD - E
    ar = jnp.arange(pad, dtype=jnp.int32)
    src2d = jnp.concatenate([src, (ar * 997) % N]).reshape(EPAD // BLK, BLK)
    dst2d = jnp.concatenate([dst, N + (ar % 128)]).reshape(EPAD // BLK, BLK)
    zeros1 = jnp.zeros((NPAD,), jnp.float32)
    xt = x.T

    deg = _deg(dst2d, zeros1).reshape(NCORES, 1, NPAD)
    dinv, u = _tc_call(_k0_body, ((1, N), (16, N)), deg, xt, W1)

    def prop(u_t, pd):
        f = _scatter16 if pd == 16 else _scatter32
        u_fm = jnp.pad(u_t, ((0, 0), (0, NPAD - N))).reshape(-1)
        s = f(src2d, dst2d, u_fm, zeros1)
        return s.reshape(NCORES, pd, NPAD)

    b = [v.reshape(-1, 1) for v in (b1, b2, b3, b4, b5, b6, b7, b8)]

    s = prop(u, 16)
    u = _tc_call(_kB_body, ((16, N),), s, u, dinv, b[0])[0]
    s = prop(u, 16)
    u = _tc_call(_kC_body, ((32, N),), s, u, dinv, W2, b[1], W3)[0]
    for bias, w, dnext in ((b[2], W4, 32), (b[3], W5, 32),
                           (b[4], W6, 32), (b[5], W7, 16)):
        s = prop(u, 32)
        u = _tc_call(_kA_body, ((dnext, N),), s, u, dinv, bias, w)[0]
    s = prop(u, 16)
    u = _tc_call(_kB_body, ((16, N),), s, u, dinv, b[6])[0]
    s = prop(u, 16)
    out_t = _tc_call(_kD_body, ((2, N),),
                     s, u, dinv, W8, b[7], xt[0:2])[0]
    return out_t.T


# speedup vs baseline: 8.4517x; 1.3900x over previous
import functools

import jax
import jax.numpy as jnp
from jax import lax
from jax.experimental import pallas as pl
from jax.experimental.pallas import tpu as pltpu
from jax.experimental.pallas import tpu_sc as plsc

N = 10000
E = 320000
NCORES = 2
NSUB = 16
NW = NCORES * NSUB
BLK = 128
NBLK = 80
EPAD = NW * NBLK * BLK
NPAD = 10240
RPT = NPAD // NSUB
LCH = 640

_MESH = plsc.VectorSubcoreMesh(core_axis_name="c", subcore_axis_name="s")


def _deg_kernel(dst2d, zeros1, deg_out, ones_v, dst_v, acc, sem):
    del sem
    cid = lax.axis_index("c")
    sid = lax.axis_index("s")
    w = cid * NSUB + sid

    for i in range(BLK // 16):
        ones_v[pl.ds(i * 16, 16)] = jnp.ones((16,), jnp.float32)
    pltpu.sync_copy(dst2d.at[pl.ds(w * NBLK, NBLK)], dst_v)
    pltpu.sync_copy(zeros1.at[pl.ds(sid * RPT, RPT)],
                    acc.at[pl.ds(sid * RPT, RPT)])
    plsc.subcore_barrier()

    def body(j, carry):
        pltpu.sync_copy(ones_v, acc.at[dst_v.at[j]], add=True)
        return carry

    lax.fori_loop(0, NBLK, body, 0)
    plsc.subcore_barrier()
    pltpu.sync_copy(acc.at[pl.ds(sid * RPT, RPT)],
                    deg_out.at[pl.ds(cid * NPAD + sid * RPT, RPT)])


_deg = functools.partial(
    pl.kernel,
    out_type=jax.ShapeDtypeStruct((NCORES * NPAD,), jnp.float32),
    mesh=_MESH,
    scratch_types=[
        pltpu.VMEM((BLK,), jnp.float32),
        pltpu.VMEM((NBLK, BLK), jnp.int32),
        pltpu.VMEM_SHARED((NPAD,), jnp.float32),
        pltpu.SemaphoreType.DMA,
    ],
)(_deg_kernel)


def _make_scatter(pd):
    @functools.partial(
        pl.kernel,
        out_type=jax.ShapeDtypeStruct((NCORES * pd * NPAD,), jnp.float32),
        mesh=_MESH,
        scratch_types=[
            pltpu.VMEM((NBLK, BLK), jnp.int32),
            pltpu.VMEM((NBLK, BLK), jnp.int32),
            [pltpu.VMEM((BLK,), jnp.float32) for _ in range(pd)],
            [pltpu.VMEM((BLK,), jnp.float32) for _ in range(pd)],
            [pltpu.VMEM_SHARED((NPAD,), jnp.float32) for _ in range(pd)],
            [pltpu.VMEM_SHARED((NPAD,), jnp.float32) for _ in range(pd)],
            pltpu.SemaphoreType.DMA,
            pltpu.SemaphoreType.DMA,
        ],
    )
    def scatter_kernel(src2d, dst2d, u_fm, zeros1, out_hbm,
                       src_v, dst_v, cola, colb, u_sc, acc, semg, sems):
        cid = lax.axis_index("c")
        sid = lax.axis_index("s")
        w = cid * NSUB + sid

        pltpu.sync_copy(src2d.at[pl.ds(w * NBLK, NBLK)], src_v)
        pltpu.sync_copy(dst2d.at[pl.ds(w * NBLK, NBLK)], dst_v)
        cps = []
        for k in range(pd):
            cps.append(pltpu.async_copy(
                u_fm.at[pl.ds(k * NPAD + sid * RPT, RPT)],
                u_sc[k].at[pl.ds(sid * RPT, RPT)], semg))
            cps.append(pltpu.async_copy(
                zeros1.at[pl.ds(sid * RPT, RPT)],
                acc[k].at[pl.ds(sid * RPT, RPT)], sems))
        for cp in cps:
            cp.wait()
        plsc.subcore_barrier()

        def body(h, carry):
            j = 2 * h
            ga = [pltpu.async_copy(u_sc[k].at[src_v.at[j]], cola[k], semg)
                  for k in range(pd)]
            gb = [pltpu.async_copy(u_sc[k].at[src_v.at[j + 1]], colb[k],
                                   semg)
                  for k in range(pd)]
            for cp in ga:
                cp.wait()
            sa = [pltpu.async_copy(cola[k], acc[k].at[dst_v.at[j]], sems,
                                   add=True)
                  for k in range(pd)]
            for cp in gb:
                cp.wait()
            sb = [pltpu.async_copy(colb[k], acc[k].at[dst_v.at[j + 1]],
                                   sems, add=True)
                  for k in range(pd)]
            for cp in sa + sb:
                cp.wait()
            return carry

        lax.fori_loop(0, NBLK // 2, body, 0)
        plsc.subcore_barrier()
        cps = [pltpu.async_copy(
                   acc[k].at[pl.ds(sid * RPT, RPT)],
                   out_hbm.at[pl.ds((cid * pd + k) * NPAD + sid * RPT, RPT)],
                   semg)
               for k in range(pd)]
        for cp in cps:
            cp.wait()

    return scatter_kernel


_scatter16 = _make_scatter(16)
_scatter32 = _make_scatter(32)


def _dotT(w, h):
    return lax.dot_general(w, h, (((0,), (0,)), ((), ())),
                           preferred_element_type=jnp.float32,
                           precision=lax.Precision.HIGHEST)


def _tc_call(body, shapes, *args):
    out_shape = [jax.ShapeDtypeStruct(s, jnp.float32) for s in shapes]
    return pl.pallas_call(body, out_shape=out_shape)(*args)


def _k0_body(deg_ref, xt_ref, w1_ref, dinv_ref, u_ref):
    deg = deg_ref[0, :, :N] + deg_ref[1, :, :N] + 1.0
    dinv = lax.rsqrt(deg)
    dinv_ref[...] = dinv
    u_ref[...] = dinv * _dotT(w1_ref[...], xt_ref[...])


def _combine(s_ref, u_ref, dinv_ref):
    return dinv_ref[...] * (s_ref[0, :, :N] + s_ref[1, :, :N] + u_ref[...])


def _kB_body(s_ref, u_ref, dinv_ref, b_ref, out_ref):
    h = jax.nn.relu(_combine(s_ref, u_ref, dinv_ref) + b_ref[...])
    out_ref[...] = dinv_ref[...] * h


def _kA_body(s_ref, u_ref, dinv_ref, b_ref, w_ref, out_ref):
    h = jax.nn.relu(_combine(s_ref, u_ref, dinv_ref) + b_ref[...])
    out_ref[...] = dinv_ref[...] * _dotT(w_ref[...], h)


def _kC_body(s_ref, u_ref, dinv_ref, w2_ref, b2_ref, w3_ref, out_ref):
    t = _combine(s_ref, u_ref, dinv_ref)
    h = jax.nn.relu(_dotT(w2_ref[...], t) + b2_ref[...])
    out_ref[...] = dinv_ref[...] * _dotT(w3_ref[...], h)


def _kD_body(s_ref, u_ref, dinv_ref, w8_ref, b8_ref, x2_ref, out_ref):
    t = _combine(s_ref, u_ref, dinv_ref)
    out_ref[...] = _dotT(w8_ref[...], t) + b8_ref[...] + x2_ref[...]


def kernel(x, edge_index, W1, W2, W3, W4, W5, W6, W7, W8,
           b1, b2, b3, b4, b5, b6, b7, b8):
    src, dst = edge_index[0], edge_index[1]
    pad = EPAD - E
    ar = jnp.arange(pad, dtype=jnp.int32)
    src2d = jnp.concatenate([src, (ar * 997) % N]).reshape(EPAD // BLK, BLK)
    dst2d = jnp.concatenate([dst, N + (ar % 128)]).reshape(EPAD // BLK, BLK)
    zeros1 = jnp.zeros((NPAD,), jnp.float32)
    xt = x.T

    deg = _deg(dst2d, zeros1).reshape(NCORES, 1, NPAD)
    dinv, u = _tc_call(_k0_body, ((1, N), (16, N)), deg, xt, W1)

    def prop(u_t, pd):
        f = _scatter16 if pd == 16 else _scatter32
        u_fm = jnp.pad(u_t, ((0, 0), (0, NPAD - N))).reshape(-1)
        s = f(src2d, dst2d, u_fm, zeros1)
        return s.reshape(NCORES, pd, NPAD)

    b = [v.reshape(-1, 1) for v in (b1, b2, b3, b4, b5, b6, b7, b8)]

    s = prop(u, 16)
    u = _tc_call(_kB_body, ((16, N),), s, u, dinv, b[0])[0]
    s = prop(u, 16)
    u = _tc_call(_kC_body, ((32, N),), s, u, dinv, W2, b[1], W3)[0]
    for bias, w, dnext in ((b[2], W4, 32), (b[3], W5, 32),
                           (b[4], W6, 32), (b[5], W7, 16)):
        s = prop(u, 32)
        u = _tc_call(_kA_body, ((dnext, N),), s, u, dinv, bias, w)[0]
    s = prop(u, 16)
    u = _tc_call(_kB_body, ((16, N),), s, u, dinv, b[6])[0]
    s = prop(u, 16)
    out_t = _tc_call(_kD_body, ((2, N),),
                     s, u, dinv, W8, b[7], xt[0:2])[0]
    return out_t.T


# speedup vs baseline: 8.4955x; 1.0052x over previous
import functools

import jax
import jax.numpy as jnp
from jax import lax
from jax.experimental import pallas as pl
from jax.experimental.pallas import tpu as pltpu
from jax.experimental.pallas import tpu_sc as plsc

N = 10000
E = 320000
NCORES = 2
NSUB = 16
NW = NCORES * NSUB
BLK = 128
NBLK = 80
EPAD = NW * NBLK * BLK
NPAD = 10240
RPT = NPAD // NSUB
LCH = 640

_MESH = plsc.VectorSubcoreMesh(core_axis_name="c", subcore_axis_name="s")


def _deg_kernel(dst2d, zeros1, deg_out, ones_v, dst_v, acc, sem):
    del sem
    cid = lax.axis_index("c")
    sid = lax.axis_index("s")
    w = cid * NSUB + sid

    for i in range(BLK // 16):
        ones_v[pl.ds(i * 16, 16)] = jnp.ones((16,), jnp.float32)
    pltpu.sync_copy(dst2d.at[pl.ds(w * NBLK, NBLK)], dst_v)
    pltpu.sync_copy(zeros1.at[pl.ds(sid * RPT, RPT)],
                    acc.at[pl.ds(sid * RPT, RPT)])
    plsc.subcore_barrier()

    def body(j, carry):
        pltpu.sync_copy(ones_v, acc.at[dst_v.at[j]], add=True)
        return carry

    lax.fori_loop(0, NBLK, body, 0)
    plsc.subcore_barrier()
    pltpu.sync_copy(acc.at[pl.ds(sid * RPT, RPT)],
                    deg_out.at[pl.ds(cid * NPAD + sid * RPT, RPT)])


_deg = functools.partial(
    pl.kernel,
    out_type=jax.ShapeDtypeStruct((NCORES * NPAD,), jnp.float32),
    mesh=_MESH,
    scratch_types=[
        pltpu.VMEM((BLK,), jnp.float32),
        pltpu.VMEM((NBLK, BLK), jnp.int32),
        pltpu.VMEM_SHARED((NPAD,), jnp.float32),
        pltpu.SemaphoreType.DMA,
    ],
)(_deg_kernel)


def _make_scatter(pd):
    @functools.partial(
        pl.kernel,
        out_type=jax.ShapeDtypeStruct((NCORES * pd * NPAD,), jnp.float32),
        mesh=_MESH,
        scratch_types=[
            pltpu.VMEM((NBLK, BLK), jnp.int32),
            pltpu.VMEM((NBLK, BLK), jnp.int32),
            [pltpu.VMEM((BLK,), jnp.float32) for _ in range(pd)],
            [pltpu.VMEM((BLK,), jnp.float32) for _ in range(pd)],
            pltpu.VMEM((2 * pd * BLK,), jnp.float32),
            [pltpu.VMEM_SHARED((NPAD,), jnp.float32) for _ in range(pd)],
            [pltpu.VMEM_SHARED((NPAD,), jnp.float32) for _ in range(pd)],
            pltpu.SemaphoreType.DMA,
            pltpu.SemaphoreType.DMA,
        ],
    )
    def scatter_kernel(src2d, dst2d, u_fm, zeros1, out_hbm,
                       src_v, dst_v, cola, colb, dummy, u_sc, acc,
                       semg, sems):
        cid = lax.axis_index("c")
        sid = lax.axis_index("s")
        w = cid * NSUB + sid

        pltpu.sync_copy(src2d.at[pl.ds(w * NBLK, NBLK)], src_v)
        pltpu.sync_copy(dst2d.at[pl.ds(w * NBLK, NBLK)], dst_v)
        cps = []
        for k in range(pd):
            cps.append(pltpu.async_copy(
                u_fm.at[pl.ds(k * NPAD + sid * RPT, RPT)],
                u_sc[k].at[pl.ds(sid * RPT, RPT)], semg))
            cps.append(pltpu.async_copy(
                zeros1.at[pl.ds(sid * RPT, RPT)],
                acc[k].at[pl.ds(sid * RPT, RPT)], sems))
        for cp in cps:
            cp.wait()
        plsc.subcore_barrier()

        def body(h, carry):
            j = 2 * h
            for k in range(pd):
                pltpu.async_copy(u_sc[k].at[src_v.at[j]], cola[k], semg)
            for k in range(pd):
                pltpu.async_copy(u_sc[k].at[src_v.at[j + 1]], colb[k], semg)
            pltpu.make_async_copy(zeros1.at[pl.ds(0, 2 * pd * BLK)],
                                  dummy, semg).wait()
            for k in range(pd):
                pltpu.async_copy(cola[k], acc[k].at[dst_v.at[j]], sems,
                                 add=True)
            for k in range(pd):
                pltpu.async_copy(colb[k], acc[k].at[dst_v.at[j + 1]], sems,
                                 add=True)
            pltpu.make_async_copy(zeros1.at[pl.ds(0, 2 * pd * BLK)],
                                  dummy, sems).wait()
            return carry

        lax.fori_loop(0, NBLK // 2, body, 0)
        plsc.subcore_barrier()
        cps = [pltpu.async_copy(
                   acc[k].at[pl.ds(sid * RPT, RPT)],
                   out_hbm.at[pl.ds((cid * pd + k) * NPAD + sid * RPT, RPT)],
                   semg)
               for k in range(pd)]
        for cp in cps:
            cp.wait()

    return scatter_kernel


_scatter16 = _make_scatter(16)
_scatter32 = _make_scatter(32)


def _dotT(w, h):
    return lax.dot_general(w, h, (((0,), (0,)), ((), ())),
                           preferred_element_type=jnp.float32,
                           precision=lax.Precision.HIGHEST)


def _tc_call(body, shapes, *args):
    out_shape = [jax.ShapeDtypeStruct(s, jnp.float32) for s in shapes]
    return pl.pallas_call(body, out_shape=out_shape)(*args)


def _k0_body(deg_ref, xt_ref, w1_ref, dinv_ref, u_ref):
    deg = deg_ref[0, :, :N] + deg_ref[1, :, :N] + 1.0
    dinv = lax.rsqrt(deg)
    dinv_ref[...] = dinv
    u_ref[...] = dinv * _dotT(w1_ref[...], xt_ref[...])


def _combine(s_ref, u_ref, dinv_ref):
    return dinv_ref[...] * (s_ref[0, :, :N] + s_ref[1, :, :N] + u_ref[...])


def _kB_body(s_ref, u_ref, dinv_ref, b_ref, out_ref):
    h = jax.nn.relu(_combine(s_ref, u_ref, dinv_ref) + b_ref[...])
    out_ref[...] = dinv_ref[...] * h


def _kA_body(s_ref, u_ref, dinv_ref, b_ref, w_ref, out_ref):
    h = jax.nn.relu(_combine(s_ref, u_ref, dinv_ref) + b_ref[...])
    out_ref[...] = dinv_ref[...] * _dotT(w_ref[...], h)


def _kC_body(s_ref, u_ref, dinv_ref, w2_ref, b2_ref, w3_ref, out_ref):
    t = _combine(s_ref, u_ref, dinv_ref)
    h = jax.nn.relu(_dotT(w2_ref[...], t) + b2_ref[...])
    out_ref[...] = dinv_ref[...] * _dotT(w3_ref[...], h)


def _kD_body(s_ref, u_ref, dinv_ref, w8_ref, b8_ref, x2_ref, out_ref):
    t = _combine(s_ref, u_ref, dinv_ref)
    out_ref[...] = _dotT(w8_ref[...], t) + b8_ref[...] + x2_ref[...]


def kernel(x, edge_index, W1, W2, W3, W4, W5, W6, W7, W8,
           b1, b2, b3, b4, b5, b6, b7, b8):
    src, dst = edge_index[0], edge_index[1]
    pad = EPAD - E
    ar = jnp.arange(pad, dtype=jnp.int32)
    src2d = jnp.concatenate([src, (ar * 997) % N]).reshape(EPAD // BLK, BLK)
    dst2d = jnp.concatenate([dst, N + (ar % 128)]).reshape(EPAD // BLK, BLK)
    zeros1 = jnp.zeros((NPAD,), jnp.float32)
    xt = x.T

    deg = _deg(dst2d, zeros1).reshape(NCORES, 1, NPAD)
    dinv, u = _tc_call(_k0_body, ((1, N), (16, N)), deg, xt, W1)

    def prop(u_t, pd):
        f = _scatter16 if pd == 16 else _scatter32
        u_fm = jnp.pad(u_t, ((0, 0), (0, NPAD - N))).reshape(-1)
        s = f(src2d, dst2d, u_fm, zeros1)
        return s.reshape(NCORES, pd, NPAD)

    b = [v.reshape(-1, 1) for v in (b1, b2, b3, b4, b5, b6, b7, b8)]

    s = prop(u, 16)
    u = _tc_call(_kB_body, ((16, N),), s, u, dinv, b[0])[0]
    s = prop(u, 16)
    u = _tc_call(_kC_body, ((32, N),), s, u, dinv, W2, b[1], W3)[0]
    for bias, w, dnext in ((b[2], W4, 32), (b[3], W5, 32),
                           (b[4], W6, 32), (b[5], W7, 16)):
        s = prop(u, 32)
        u = _tc_call(_kA_body, ((dnext, N),), s, u, dinv, bias, w)[0]
    s = prop(u, 16)
    u = _tc_call(_kB_body, ((16, N),), s, u, dinv, b[6])[0]
    s = prop(u, 16)
    out_t = _tc_call(_kD_body, ((2, N),),
                     s, u, dinv, W8, b[7], xt[0:2])[0]
    return out_t.T
